# Initial kernel scaffold; baseline (speedup 1.0000x reference)
import jax
import jax.numpy as jnp
from jax.experimental import pallas as pl


def kernel(x, nodes, feats, neigh0, neigh1, W0, b0, W1, b1):
    raise NotImplementedError("write your pallas kernel here")


# speedup vs baseline: 2.2442x; 2.2442x over previous
import functools

import jax
import jax.numpy as jnp
from jax import lax
from jax.experimental import pallas as pl
from jax.experimental.pallas import tpu as pltpu
from jax.experimental.pallas import tpu_sc as plsc

LANES = 16


@functools.lru_cache(maxsize=None)
def _make_gather_sum(n_nodes, d, b, fan, nc, ns):
    nw = nc * ns
    b_per_w = b // nw
    sb = max(1, 128 // fan)
    while b_per_w % sb:
        sb -= 1
    n_chunks = b_per_w // sb
    idxw = sb * fan

    mesh = plsc.VectorSubcoreMesh(core_axis_name="c", subcore_axis_name="s")

    @functools.partial(
        pl.kernel,
        mesh=mesh,
        out_type=jax.ShapeDtypeStruct((b, d), jnp.float32),
        scratch_types=[
            pltpu.VMEM((n_chunks, idxw), jnp.int32),
            pltpu.VMEM((idxw, d), jnp.float32),
            pltpu.VMEM((b_per_w, d), jnp.float32),
            pltpu.SemaphoreType.DMA,
        ],
    )
    def gather_sum(neigh_hbm, feats_hbm, out_hbm, idx_v, buf_v, acc_v, sem):
        cid = lax.axis_index("c")
        sid = lax.axis_index("s")
        wid = sid * nc + cid
        pltpu.sync_copy(neigh_hbm.at[pl.ds(wid * n_chunks, n_chunks)], idx_v)

        def chunk_body(k, carry):
            pltpu.async_copy(feats_hbm.at[idx_v.at[k]], buf_v, sem).wait()

            def d_body(dst, carry2):
                for c in range(d // LANES):
                    cs = pl.ds(c * LANES, LANES)
                    v = buf_v[dst * fan, cs]
                    for j in range(1, fan):
                        v = v + buf_v[dst * fan + j, cs]
                    acc_v[k * sb + dst, cs] = v
                return carry2

            lax.fori_loop(0, sb, d_body, 0)
            return carry

        lax.fori_loop(0, n_chunks, chunk_body, 0)
        pltpu.sync_copy(acc_v, out_hbm.at[pl.ds(wid * b_per_w, b_per_w)])

    return gather_sum


def _tc_body(inv, x_ref, s_ref, w_ref, b_ref, o_ref):
    x = x_ref[...]
    agg = (x + s_ref[...]) * inv
    h = jnp.concatenate([x, agg], axis=1)
    acc = lax.dot_general(
        h, w_ref[...], (((1,), (1,)), ((), ())),
        preferred_element_type=jnp.float32)
    o_ref[...] = jnp.maximum(acc + b_ref[...], 0.0)


@functools.lru_cache(maxsize=None)
def _make_fused_linear(b, d, h, fan, bm):
    grid = (b // bm,)
    return pl.pallas_call(
        functools.partial(_tc_body, 1.0 / (fan + 1)),
        grid=grid,
        in_specs=[
            pl.BlockSpec((bm, d), lambda i: (i, 0)),
            pl.BlockSpec((bm, d), lambda i: (i, 0)),
            pl.BlockSpec((h, 2 * d), lambda i: (0, 0)),
            pl.BlockSpec((1, h), lambda i: (0, 0)),
        ],
        out_specs=pl.BlockSpec((bm, h), lambda i: (i, 0)),
        out_shape=jax.ShapeDtypeStruct((b, h), jnp.float32),
    )


def kernel(x, nodes, feats, neigh0, neigh1, W0, b0, W1, b1):
    b, d = x.shape
    fan = neigh1.shape[1]
    n_nodes = feats.shape[0]
    h = W1.shape[0]

    info = plsc.get_sparse_core_info()
    nc, ns = info.num_cores, info.num_subcores
    nw = nc * ns
    b_per_w = b // nw
    sb = max(1, 128 // fan)
    while b_per_w % sb:
        sb -= 1
    idxw = sb * fan

    neigh_rows = neigh1.reshape(b * fan // idxw, idxw)
    gather_sum = _make_gather_sum(n_nodes, d, b, fan, nc, ns)
    sums = gather_sum(neigh_rows, feats)

    fused = _make_fused_linear(b, d, h, fan, 512)
    out = fused(x, sums, W1, b1.reshape(1, h))
    return out[:, None, :]


# speedup vs baseline: 2.7951x; 1.2455x over previous
import functools

import jax
import jax.numpy as jnp
from jax import lax
from jax.experimental import pallas as pl
from jax.experimental.pallas import tpu as pltpu
from jax.experimental.pallas import tpu_sc as plsc

LANES = 16


@functools.lru_cache(maxsize=None)
def _make_gather_sum(n_nodes, d, b, fan, nc, ns):
    nw = nc * ns
    b_per_w = b // nw
    sb = max(1, 128 // fan)
    while b_per_w % sb:
        sb -= 1
    n_chunks = b_per_w // sb
    idxw = sb * fan

    mesh = plsc.VectorSubcoreMesh(core_axis_name="c", subcore_axis_name="s")

    assert n_chunks % 2 == 0

    @functools.partial(
        pl.kernel,
        mesh=mesh,
        out_type=jax.ShapeDtypeStruct((b, d), jnp.float32),
        scratch_types=[
            pltpu.VMEM((n_chunks, idxw), jnp.int32),
            pltpu.VMEM((idxw, d), jnp.float32),
            pltpu.VMEM((idxw, d), jnp.float32),
            pltpu.VMEM((b_per_w, d), jnp.float32),
            pltpu.SemaphoreType.DMA,
            pltpu.SemaphoreType.DMA,
        ],
    )
    def gather_sum(neigh_hbm, feats_hbm, out_hbm, idx_v, buf0, buf1, acc_v,
                   sem0, sem1):
        cid = lax.axis_index("c")
        sid = lax.axis_index("s")
        wid = sid * nc + cid
        pltpu.sync_copy(neigh_hbm.at[pl.ds(wid * n_chunks, n_chunks)], idx_v)
        pltpu.async_copy(feats_hbm.at[idx_v.at[0]], buf0, sem0)

        def acc_chunk(k, buf):
            def d_body(dst, carry2):
                for c in range(d // LANES):
                    cs = pl.ds(c * LANES, LANES)
                    v = buf[dst * fan, cs]
                    for j in range(1, fan):
                        v = v + buf[dst * fan + j, cs]
                    acc_v[k * sb + dst, cs] = v
                return carry2

            lax.fori_loop(0, sb, d_body, 0)

        def pair_body(i, carry):
            k0 = 2 * i
            pltpu.async_copy(feats_hbm.at[idx_v.at[k0 + 1]], buf1, sem1)
            pltpu.make_async_copy(feats_hbm.at[idx_v.at[0]], buf0, sem0).wait()
            acc_chunk(k0, buf0)

            @pl.when(k0 + 2 < n_chunks)
            def _():
                pltpu.async_copy(feats_hbm.at[idx_v.at[k0 + 2]], buf0, sem0)

            pltpu.make_async_copy(feats_hbm.at[idx_v.at[0]], buf1, sem1).wait()
            acc_chunk(k0 + 1, buf1)
            return carry

        lax.fori_loop(0, n_chunks // 2, pair_body, 0)
        pltpu.sync_copy(acc_v, out_hbm.at[pl.ds(wid * b_per_w, b_per_w)])

    return gather_sum


def _tc_body(inv, x_ref, s_ref, w_ref, b_ref, o_ref):
    x = x_ref[...]
    agg = (x + s_ref[...]) * inv
    h = jnp.concatenate([x, agg], axis=1)
    acc = lax.dot_general(
        h, w_ref[...], (((1,), (1,)), ((), ())),
        preferred_element_type=jnp.float32)
    o_ref[...] = jnp.maximum(acc + b_ref[...], 0.0)


@functools.lru_cache(maxsize=None)
def _make_fused_linear(b, d, h, fan, bm):
    grid = (b // bm,)
    return pl.pallas_call(
        functools.partial(_tc_body, 1.0 / (fan + 1)),
        grid=grid,
        in_specs=[
            pl.BlockSpec((bm, d), lambda i: (i, 0)),
            pl.BlockSpec((bm, d), lambda i: (i, 0)),
            pl.BlockSpec((h, 2 * d), lambda i: (0, 0)),
            pl.BlockSpec((1, h), lambda i: (0, 0)),
        ],
        out_specs=pl.BlockSpec((bm, h), lambda i: (i, 0)),
        out_shape=jax.ShapeDtypeStruct((b, h), jnp.float32),
    )


def kernel(x, nodes, feats, neigh0, neigh1, W0, b0, W1, b1):
    b, d = x.shape
    fan = neigh1.shape[1]
    n_nodes = feats.shape[0]
    h = W1.shape[0]

    info = plsc.get_sparse_core_info()
    nc, ns = info.num_cores, info.num_subcores
    nw = nc * ns
    b_per_w = b // nw
    sb = max(1, 128 // fan)
    while b_per_w % sb:
        sb -= 1
    idxw = sb * fan

    neigh_rows = neigh1.reshape(b * fan // idxw, idxw)
    gather_sum = _make_gather_sum(n_nodes, d, b, fan, nc, ns)
    sums = gather_sum(neigh_rows, feats)

    fused = _make_fused_linear(b, d, h, fan, 512)
    out = fused(x, sums, W1, b1.reshape(1, h))
    return out[:, None, :]
